# baseline (speedup 1.0000x reference)
import jax
import jax.numpy as jnp
from jax import lax
from jax.experimental import pallas as pl
from jax.experimental.pallas import tpu as pltpu
from jax.experimental.pallas import tpu_sc as plsc

L1, L2, ORBIT, DIM = 4, 3, 2, 2
N_SITES = L1 * L2 * ORBIT
TABLE_SIZE = DIM ** N_SITES

NC, NS, LANES = 2, 16, 16
NW = NC * NS
BATCH = 16384
B_PER_W = BATCH // NW
CHUNK = 128
N_CHUNK = B_PER_W // CHUNK
GROUPS_PER_CHUNK = CHUNK // LANES


def _site(j):
    return (j // 6, (j % 6) // 2, j % 2)


def _wf_body(xr_hbm, real_hbm, imag_hbm, out_r_hbm, out_i_hbm,
             x_v, idx_v, outr_v, outi_v,
             sem_x0, sem_x1, sem_x2, sem_x3, sem_r, sem_i):
    wid = lax.axis_index("s") * NC + lax.axis_index("c")
    base = wid * B_PER_W
    sems = (sem_x0, sem_x1, sem_x2, sem_x3)

    x_copies = [
        pltpu.async_copy(
            xr_hbm.at[:, :, :, pl.ds(base + ch * CHUNK, CHUNK)],
            x_v.at[:, :, :, pl.ds(ch * CHUNK, CHUNK)],
            sems[ch])
        for ch in range(N_CHUNK)
    ]

    g_copies = []
    for ch in range(N_CHUNK):
        x_copies[ch].wait()

        def body(g, carry, ch=ch):
            start = ch * CHUNK + g * LANES
            acc = jnp.zeros((LANES,), jnp.int32)
            for j in range(N_SITES):
                a, c, d = _site(j)
                acc = acc + acc + x_v[a, c, d, pl.ds(start, LANES)]
            idx_v[pl.ds(start, LANES)] = acc
            return carry

        lax.fori_loop(0, GROUPS_PER_CHUNK, body, 0)

        sl = pl.ds(ch * CHUNK, CHUNK)
        g_copies.append(
            pltpu.async_copy(real_hbm.at[idx_v.at[sl]], outr_v.at[sl], sem_r))
        g_copies.append(
            pltpu.async_copy(imag_hbm.at[idx_v.at[sl]], outi_v.at[sl], sem_i))

    for cp in g_copies:
        cp.wait()

    bsl = pl.ds(base, B_PER_W)
    pltpu.sync_copy(outr_v, out_r_hbm.at[bsl])
    pltpu.sync_copy(outi_v, out_i_hbm.at[bsl])


_wf = pl.kernel(
    _wf_body,
    mesh=plsc.VectorSubcoreMesh(core_axis_name="c", subcore_axis_name="s"),
    out_type=[
        jax.ShapeDtypeStruct((BATCH,), jnp.float32),
        jax.ShapeDtypeStruct((BATCH,), jnp.float32),
    ],
    scratch_types=[
        pltpu.VMEM((L1, L2, ORBIT, B_PER_W), jnp.int32),
        pltpu.VMEM((B_PER_W,), jnp.int32),
        pltpu.VMEM((B_PER_W,), jnp.float32),
        pltpu.VMEM((B_PER_W,), jnp.float32),
        pltpu.SemaphoreType.DMA,
        pltpu.SemaphoreType.DMA,
        pltpu.SemaphoreType.DMA,
        pltpu.SemaphoreType.DMA,
        pltpu.SemaphoreType.DMA,
        pltpu.SemaphoreType.DMA,
    ],
)


def kernel(x, wave_real, wave_imag):
    lead = x.shape[:-3]
    xr = x.reshape((-1,) + x.shape[-3:]).transpose(1, 2, 3, 0)
    out_r, out_i = _wf(xr, wave_real, wave_imag)
    one = jnp.float32(1) + wave_real[0] * jnp.float32(0)
    return lax.complex(out_r * one, out_i * one).reshape(lead)


# speedup vs baseline: 1.0074x; 1.0074x over previous
import jax
import jax.numpy as jnp
from jax import lax
from jax.experimental import pallas as pl
from jax.experimental.pallas import tpu as pltpu
from jax.experimental.pallas import tpu_sc as plsc

L1, L2, ORBIT, DIM = 4, 3, 2, 2
N_SITES = L1 * L2 * ORBIT
TABLE_SIZE = DIM ** N_SITES

NC, NS, LANES = 2, 16, 16
NW = NC * NS
BATCH = 16384
B_PER_W = BATCH // NW
CHUNK = 128
N_CHUNK = B_PER_W // CHUNK
N_GROUPS = B_PER_W // LANES
HALF_SITES = N_SITES // 2


def _site(j):
    return (j // 6, (j % 6) // 2, j % 2)


def _wf_body(xr_hbm, real_hbm, imag_hbm, out_r_hbm, out_i_hbm,
             x_v, idx_v, outr_v, outi_v, sem_x, sem_r, sem_i):
    wid = lax.axis_index("s") * NC + lax.axis_index("c")
    base = wid * B_PER_W
    bsl = pl.ds(base, B_PER_W)

    cp1 = pltpu.async_copy(xr_hbm.at[pl.ds(0, 2), :, :, bsl],
                           x_v.at[pl.ds(0, 2)], sem_x)
    cp2 = pltpu.async_copy(xr_hbm.at[pl.ds(2, 2), :, :, bsl],
                           x_v.at[pl.ds(2, 2)], sem_x)
    cp1.wait()

    def body1(g, carry):
        start = g * LANES
        acc = jnp.zeros((LANES,), jnp.int32)
        for j in range(HALF_SITES):
            a, c, d = _site(j)
            bits = x_v[a, c, d, pl.ds(start, LANES)]
            acc = acc + jnp.left_shift(bits, N_SITES - 1 - j)
        idx_v[pl.ds(start, LANES)] = acc
        return carry

    lax.fori_loop(0, N_GROUPS, body1, 0)
    cp2.wait()

    g_copies = []
    for ch in range(N_CHUNK):
        def body2(g, carry, ch=ch):
            start = ch * CHUNK + g * LANES
            acc = idx_v[pl.ds(start, LANES)]
            for j in range(HALF_SITES, N_SITES):
                a, c, d = _site(j)
                bits = x_v[a, c, d, pl.ds(start, LANES)]
                acc = acc + jnp.left_shift(bits, N_SITES - 1 - j)
            idx_v[pl.ds(start, LANES)] = acc
            return carry

        lax.fori_loop(0, CHUNK // LANES, body2, 0)
        sl = pl.ds(ch * CHUNK, CHUNK)
        g_copies.append(
            pltpu.async_copy(real_hbm.at[idx_v.at[sl]], outr_v.at[sl], sem_r))
        g_copies.append(
            pltpu.async_copy(imag_hbm.at[idx_v.at[sl]], outi_v.at[sl], sem_i))
    for cp in g_copies:
        cp.wait()

    pltpu.sync_copy(outr_v, out_r_hbm.at[bsl])
    pltpu.sync_copy(outi_v, out_i_hbm.at[bsl])


_wf = pl.kernel(
    _wf_body,
    mesh=plsc.VectorSubcoreMesh(core_axis_name="c", subcore_axis_name="s"),
    out_type=[
        jax.ShapeDtypeStruct((BATCH,), jnp.float32),
        jax.ShapeDtypeStruct((BATCH,), jnp.float32),
    ],
    scratch_types=[
        pltpu.VMEM((L1, L2, ORBIT, B_PER_W), jnp.int32),
        pltpu.VMEM((B_PER_W,), jnp.int32),
        pltpu.VMEM((B_PER_W,), jnp.float32),
        pltpu.VMEM((B_PER_W,), jnp.float32),
        pltpu.SemaphoreType.DMA,
        pltpu.SemaphoreType.DMA,
        pltpu.SemaphoreType.DMA,
    ],
)


def kernel(x, wave_real, wave_imag):
    lead = x.shape[:-3]
    xr = x.reshape((-1,) + x.shape[-3:]).transpose(1, 2, 3, 0)
    out_r, out_i = _wf(xr, wave_real, wave_imag)
    one = jnp.float32(1) + wave_real[0] * jnp.float32(0)
    return lax.complex(out_r * one, out_i * one).reshape(lead)
